# Initial kernel scaffold; baseline (speedup 1.0000x reference)
import jax
import jax.numpy as jnp
from jax.experimental import pallas as pl


def kernel(seq_len, bolus_mask, dominant_drugs, base_frequencies, drug_freq_modulation, bolus_embedding):
    raise NotImplementedError("write your pallas kernel here")


# speedup vs baseline: 5.9630x; 5.9630x over previous
import jax
import jax.numpy as jnp
from jax import lax
from jax.experimental import pallas as pl
from jax.experimental.pallas import tpu as pltpu

_HIGH = lax.Precision.HIGHEST


def _encoding_body(nd, batch, drugs_ref, bf_ref, mod_ref, t_ref, out_ref):
    L = drugs_ref.shape[1]
    half = mod_ref.shape[1]
    drugs = drugs_ref[...]
    rows = [
        jnp.sum((drugs == d).astype(jnp.float32), axis=0, keepdims=True)
        for d in range(nd)
    ]
    counts = jnp.concatenate(rows, axis=0)
    acc = lax.dot_general(counts, mod_ref[...], (((0,), (0,)), ((), ())),
                          preferred_element_type=jnp.float32, precision=_HIGH)
    row_i = lax.broadcasted_iota(jnp.int32, (L, L), 0)
    col_i = lax.broadcasted_iota(jnp.int32, (L, L), 1)
    tri = (col_i <= row_i).astype(jnp.float32)
    csum = lax.dot_general(tri, acc, (((1,), (0,)), ((), ())),
                           preferred_element_type=jnp.float32, precision=_HIGH)
    t = t_ref[...]
    causal = jnp.where(t > 0.0, csum / ((t + 1.0) * batch), 0.0)
    ang = t * (bf_ref[...] * (1.0 + causal))
    s = jnp.sin(ang)
    c = jnp.cos(ang)
    k2 = lax.broadcasted_iota(jnp.int32, (half, 2 * half), 0) * 2
    cc = lax.broadcasted_iota(jnp.int32, (half, 2 * half), 1)
    spread_even = (cc == k2).astype(jnp.float32)
    spread_odd = (cc == k2 + 1).astype(jnp.float32)
    out_ref[...] = (
        lax.dot_general(s, spread_even, (((1,), (0,)), ((), ())),
                        preferred_element_type=jnp.float32, precision=_HIGH)
        + lax.dot_general(c, spread_odd, (((1,), (0,)), ((), ())),
                          preferred_element_type=jnp.float32, precision=_HIGH)
    )


def _broadcast_body(enc_ref, mask_ref, emb_ref, out_ref):
    out_ref[...] = enc_ref[...][None, :, :] + mask_ref[...] * emb_ref[...]


def kernel(seq_len, bolus_mask, dominant_drugs, base_frequencies,
           drug_freq_modulation, bolus_embedding):
    B, L = bolus_mask.shape
    nd, half = drug_freq_modulation.shape
    E = 2 * half

    drugs = dominant_drugs.astype(jnp.int32)
    t_idx = (jnp.arange(L, dtype=jnp.float32)
             + (jnp.asarray(seq_len) - L).astype(jnp.float32))
    t_col = t_idx[:, None]
    bf_row = base_frequencies.reshape(1, half)

    import functools
    enc = pl.pallas_call(
        functools.partial(_encoding_body, nd, float(B)),
        out_shape=jax.ShapeDtypeStruct((L, E), jnp.float32),
    )(drugs, bf_row, drug_freq_modulation, t_col)

    mask_f = bolus_mask.astype(jnp.float32)[:, :, None]
    emb = bolus_embedding.reshape(1, 1, E)

    TB = 64
    out = pl.pallas_call(
        _broadcast_body,
        grid=(B // TB,),
        in_specs=[
            pl.BlockSpec((L, E), lambda i: (0, 0)),
            pl.BlockSpec((TB, L, 1), lambda i: (i, 0, 0)),
            pl.BlockSpec((1, 1, E), lambda i: (0, 0, 0)),
        ],
        out_specs=pl.BlockSpec((TB, L, E), lambda i: (i, 0, 0)),
        out_shape=jax.ShapeDtypeStruct((B, L, E), jnp.float32),
    )(enc, mask_f, emb)
    return out


# speedup vs baseline: 15.0308x; 2.5207x over previous
import jax
import jax.numpy as jnp
from jax import lax
from jax.experimental import pallas as pl
from jax.experimental.pallas import tpu as pltpu

_HIGH = lax.Precision.HIGHEST


def _encoding_body(nd, batch, drugs_ref, bf_ref, mod_ref, t_ref, out_ref):
    L = drugs_ref.shape[1]
    half = mod_ref.shape[1]
    drugs = drugs_ref[...]
    rows = [
        jnp.sum((drugs == d).astype(jnp.float32), axis=0, keepdims=True)
        for d in range(nd)
    ]
    counts = jnp.concatenate(rows, axis=0)
    acc = lax.dot_general(counts, mod_ref[...], (((0,), (0,)), ((), ())),
                          preferred_element_type=jnp.float32, precision=_HIGH)
    row_i = lax.broadcasted_iota(jnp.int32, (L, L), 0)
    col_i = lax.broadcasted_iota(jnp.int32, (L, L), 1)
    tri = (col_i <= row_i).astype(jnp.float32)
    csum = lax.dot_general(tri, acc, (((1,), (0,)), ((), ())),
                           preferred_element_type=jnp.float32, precision=_HIGH)
    t = t_ref[...]
    causal = jnp.where(t > 0.0, csum / ((t + 1.0) * batch), 0.0)
    ang = t * (bf_ref[...] * (1.0 + causal))
    s = jnp.sin(ang)
    c = jnp.cos(ang)
    k2 = lax.broadcasted_iota(jnp.int32, (half, 2 * half), 0) * 2
    cc = lax.broadcasted_iota(jnp.int32, (half, 2 * half), 1)
    spread_even = (cc == k2).astype(jnp.float32)
    spread_odd = (cc == k2 + 1).astype(jnp.float32)
    out_ref[...] = (
        lax.dot_general(s, spread_even, (((1,), (0,)), ((), ())),
                        preferred_element_type=jnp.float32, precision=_HIGH)
        + lax.dot_general(c, spread_odd, (((1,), (0,)), ((), ())),
                          preferred_element_type=jnp.float32, precision=_HIGH)
    )


def _broadcast_body(enc_ref, maskt_ref, emb_ref, out_ref):
    enc = enc_ref[...]
    emb = emb_ref[...]
    tb = out_ref.shape[0]
    for i in range(tb):
        col = maskt_ref[:, i:i + 1]
        out_ref[i, :, :] = enc + col * emb


def kernel(seq_len, bolus_mask, dominant_drugs, base_frequencies,
           drug_freq_modulation, bolus_embedding):
    B, L = bolus_mask.shape
    nd, half = drug_freq_modulation.shape
    E = 2 * half

    drugs = dominant_drugs.astype(jnp.int32)
    t_idx = (jnp.arange(L, dtype=jnp.float32)
             + (jnp.asarray(seq_len) - L).astype(jnp.float32))
    t_col = t_idx[:, None]
    bf_row = base_frequencies.reshape(1, half)

    import functools
    enc = pl.pallas_call(
        functools.partial(_encoding_body, nd, float(B)),
        out_shape=jax.ShapeDtypeStruct((L, E), jnp.float32),
    )(drugs, bf_row, drug_freq_modulation, t_col)

    mask_t = bolus_mask.T.astype(jnp.float32)
    emb = bolus_embedding.reshape(1, E)

    TB = 128
    out = pl.pallas_call(
        _broadcast_body,
        grid=(B // TB,),
        in_specs=[
            pl.BlockSpec((L, E), lambda i: (0, 0)),
            pl.BlockSpec((L, TB), lambda i: (0, i)),
            pl.BlockSpec((1, E), lambda i: (0, 0)),
        ],
        out_specs=pl.BlockSpec((TB, L, E), lambda i: (i, 0, 0)),
        out_shape=jax.ShapeDtypeStruct((B, L, E), jnp.float32),
    )(enc, mask_t, emb)
    return out
